# Initial kernel scaffold; baseline (speedup 1.0000x reference)
import functools

import jax
import jax.numpy as jnp
from jax import lax
from jax.experimental import pallas as pl
from jax.experimental.pallas import tpu as pltpu
from jax.experimental.pallas import tpu_sc as plsc

B = 16384
F = 26
D = 32
LANES = 16

CHUNK = 64
IDX_W = 128
IDX_ROWS = CHUNK * F // IDX_W


def _make_kernel(num_workers):
    rows_per_w = B // num_workers
    nchunks = rows_per_w // CHUNK
    idx_rows_per_w = rows_per_w * F // IDX_W

    mesh = plsc.VectorSubcoreMesh(core_axis_name="c", subcore_axis_name="s")

    @functools.partial(
        pl.kernel,
        mesh=mesh,
        out_type=jax.ShapeDtypeStruct((B, D), jnp.float32),
        scratch_types=[
            pltpu.VMEM((IDX_ROWS, IDX_W), jnp.int32),
            pltpu.VMEM((CHUNK * F, D), jnp.float32),
            pltpu.VMEM((CHUNK, D), jnp.float32),
            pltpu.SemaphoreType.DMA,
        ],
    )
    def emb_kernel(feats_hbm, table_hbm, out_hbm, idx_v, rows_v, out_v, sem):
        num_cores = lax.axis_size("c")
        wid = lax.axis_index("s") * num_cores + lax.axis_index("c")

        for k in range(nchunks):
            irow0 = wid * idx_rows_per_w + k * IDX_ROWS
            pltpu.sync_copy(feats_hbm.at[pl.ds(irow0, IDX_ROWS)], idx_v)
            copies = []
            for j in range(IDX_ROWS):
                copies.append(
                    pltpu.async_copy(
                        table_hbm.at[idx_v.at[j]],
                        rows_v.at[pl.ds(j * IDX_W, IDX_W)],
                        sem,
                    )
                )
            for c in copies:
                c.wait()

            def reduce_row(c, _):
                r0 = c * F
                lo = rows_v[r0, pl.ds(0, LANES)]
                hi = rows_v[r0, pl.ds(LANES, LANES)]
                for f in range(1, F):
                    lo = lo + rows_v[r0 + f, pl.ds(0, LANES)]
                    hi = hi + rows_v[r0 + f, pl.ds(LANES, LANES)]
                out_v[c, pl.ds(0, LANES)] = lo
                out_v[c, pl.ds(LANES, LANES)] = hi
                return _

            lax.fori_loop(0, CHUNK, reduce_row, None)

            base = wid * rows_per_w + k * CHUNK
            pltpu.sync_copy(out_v, out_hbm.at[pl.ds(base, CHUNK)])

    return emb_kernel


def kernel(categorical_feats, table):
    info = plsc.get_sparse_core_info()
    num_workers = info.num_cores * info.num_subcores
    feats2d = categorical_feats.reshape(B * F // IDX_W, IDX_W).astype(jnp.int32)
    return _make_kernel(num_workers)(feats2d, table)


# speedup vs baseline: 1.9155x; 1.9155x over previous
import functools

import jax
import jax.numpy as jnp
from jax import lax
from jax.experimental import pallas as pl
from jax.experimental.pallas import tpu as pltpu
from jax.experimental.pallas import tpu_sc as plsc

B = 16384
F = 26
D = 32
LANES = 16

CHUNK = 64
IDX_W = 128
IDX_ROWS = CHUNK * F // IDX_W


def _make_kernel(num_workers):
    rows_per_w = B // num_workers
    nchunks = rows_per_w // CHUNK
    idx_rows_per_w = rows_per_w * F // IDX_W

    mesh = plsc.VectorSubcoreMesh(core_axis_name="c", subcore_axis_name="s")

    @functools.partial(
        pl.kernel,
        mesh=mesh,
        out_type=jax.ShapeDtypeStruct((B, D), jnp.float32),
        compiler_params=pltpu.CompilerParams(use_tc_tiling_on_sc=False),
        scratch_types=[
            pltpu.VMEM((IDX_ROWS * IDX_W,), jnp.int32),
            pltpu.VMEM((CHUNK * F, D), jnp.float32),
            pltpu.VMEM((CHUNK, D), jnp.float32),
            pltpu.SemaphoreType.DMA,
        ],
    )
    def emb_kernel(feats_hbm, table_hbm, out_hbm, idx_v, rows_v, out_v, sem):
        num_cores = lax.axis_size("c")
        wid = lax.axis_index("s") * num_cores + lax.axis_index("c")

        for k in range(nchunks):
            ioff = (wid * idx_rows_per_w + k * IDX_ROWS) * IDX_W
            pltpu.sync_copy(feats_hbm.at[pl.ds(ioff, IDX_ROWS * IDX_W)], idx_v)
            copies = []
            for j in range(IDX_ROWS):
                copies.append(
                    pltpu.async_copy(
                        table_hbm.at[idx_v.at[pl.ds(j * IDX_W, IDX_W)]],
                        rows_v.at[pl.ds(j * IDX_W, IDX_W)],
                        sem,
                    )
                )
            for c in copies:
                c.wait()

            def reduce_row(c, _):
                r0 = c * F
                lo = rows_v[r0, pl.ds(0, LANES)]
                hi = rows_v[r0, pl.ds(LANES, LANES)]
                for f in range(1, F):
                    lo = lo + rows_v[r0 + f, pl.ds(0, LANES)]
                    hi = hi + rows_v[r0 + f, pl.ds(LANES, LANES)]
                out_v[c, pl.ds(0, LANES)] = lo
                out_v[c, pl.ds(LANES, LANES)] = hi
                return _

            lax.fori_loop(0, CHUNK, reduce_row, None)

            base = wid * rows_per_w + k * CHUNK
            pltpu.sync_copy(out_v, out_hbm.at[pl.ds(base, CHUNK)])

    return emb_kernel


def kernel(categorical_feats, table):
    info = plsc.get_sparse_core_info()
    num_workers = info.num_cores * info.num_subcores
    feats_flat = categorical_feats.reshape(B * F).astype(jnp.int32)
    return _make_kernel(num_workers)(feats_flat, table)


# speedup vs baseline: 1.9870x; 1.0373x over previous
import functools

import jax
import jax.numpy as jnp
from jax import lax
from jax.experimental import pallas as pl
from jax.experimental.pallas import tpu as pltpu
from jax.experimental.pallas import tpu_sc as plsc

B = 16384
F = 26
D = 32
LANES = 16

CHUNK = 64
IDX_W = 128
IDX_ROWS = CHUNK * F // IDX_W


def _make_kernel(num_workers):
    rows_per_w = B // num_workers
    nchunks = rows_per_w // CHUNK
    idx_per_w = rows_per_w * F

    mesh = plsc.VectorSubcoreMesh(core_axis_name="c", subcore_axis_name="s")

    @functools.partial(
        pl.kernel,
        mesh=mesh,
        out_type=jax.ShapeDtypeStruct((B, D), jnp.float32),
        compiler_params=pltpu.CompilerParams(use_tc_tiling_on_sc=False),
        scratch_types=[
            pltpu.VMEM((idx_per_w,), jnp.int32),
            pltpu.VMEM((CHUNK * F, D), jnp.float32),
            pltpu.VMEM((CHUNK * F, D), jnp.float32),
            pltpu.VMEM((CHUNK, D), jnp.float32),
            pltpu.SemaphoreType.DMA,
            pltpu.SemaphoreType.DMA,
        ],
    )
    def emb_kernel(feats_hbm, table_hbm, out_hbm, idx_v, rows0, rows1, out_v,
                   sem0, sem1):
        num_cores = lax.axis_size("c")
        wid = lax.axis_index("s") * num_cores + lax.axis_index("c")

        pltpu.sync_copy(feats_hbm.at[pl.ds(wid * idx_per_w, idx_per_w)], idx_v)

        bufs = (rows0, rows1)
        sems = (sem0, sem1)

        def fire(k):
            buf = bufs[k % 2]
            sem = sems[k % 2]
            cs = []
            for j in range(IDX_ROWS):
                cs.append(
                    pltpu.async_copy(
                        table_hbm.at[idx_v.at[pl.ds(k * CHUNK * F + j * IDX_W,
                                                    IDX_W)]],
                        buf.at[pl.ds(j * IDX_W, IDX_W)],
                        sem,
                    )
                )
            return cs

        inflight = fire(0)
        for k in range(nchunks):
            buf = bufs[k % 2]
            nxt = fire(k + 1) if k + 1 < nchunks else []
            for c in inflight:
                c.wait()
            inflight = nxt

            def reduce_row(c, _):
                r0 = c * F
                lo = buf[r0, pl.ds(0, LANES)]
                hi = buf[r0, pl.ds(LANES, LANES)]
                for f in range(1, F):
                    lo = lo + buf[r0 + f, pl.ds(0, LANES)]
                    hi = hi + buf[r0 + f, pl.ds(LANES, LANES)]
                out_v[c, pl.ds(0, LANES)] = lo
                out_v[c, pl.ds(LANES, LANES)] = hi
                return _

            lax.fori_loop(0, CHUNK, reduce_row, None)

            base = wid * rows_per_w + k * CHUNK
            pltpu.sync_copy(out_v, out_hbm.at[pl.ds(base, CHUNK)])

    return emb_kernel


def kernel(categorical_feats, table):
    info = plsc.get_sparse_core_info()
    num_workers = info.num_cores * info.num_subcores
    feats_flat = categorical_feats.reshape(B * F).astype(jnp.int32)
    return _make_kernel(num_workers)(feats_flat, table)


# speedup vs baseline: 1.9970x; 1.0051x over previous
import functools

import jax
import jax.numpy as jnp
from jax import lax
from jax.experimental import pallas as pl
from jax.experimental.pallas import tpu as pltpu
from jax.experimental.pallas import tpu_sc as plsc

B = 16384
F = 26
D = 32
LANES = 16

CHUNK = 64


def _make_kernel(num_workers):
    rows_per_w = B // num_workers
    nchunks = rows_per_w // CHUNK

    mesh = plsc.VectorSubcoreMesh(core_axis_name="c", subcore_axis_name="s")

    @functools.partial(
        pl.kernel,
        mesh=mesh,
        out_type=jax.ShapeDtypeStruct((B, D), jnp.float32),
        compiler_params=pltpu.CompilerParams(use_tc_tiling_on_sc=False),
        scratch_types=[
            pltpu.VMEM((F, rows_per_w), jnp.int32),
            pltpu.VMEM((F, CHUNK, D), jnp.float32),
            pltpu.VMEM((F, CHUNK, D), jnp.float32),
            pltpu.VMEM((CHUNK, D), jnp.float32),
            pltpu.SemaphoreType.DMA,
            pltpu.SemaphoreType.DMA,
        ],
    )
    def emb_kernel(feats_hbm, table_hbm, out_hbm, idx_v, rows0, rows1, out_v,
                   sem0, sem1):
        num_cores = lax.axis_size("c")
        wid = lax.axis_index("s") * num_cores + lax.axis_index("c")
        b0 = wid * rows_per_w

        pltpu.sync_copy(feats_hbm.at[:, pl.ds(b0, rows_per_w)], idx_v)

        bufs = (rows0, rows1)
        sems = (sem0, sem1)

        def fire(k):
            buf = bufs[k % 2]
            sem = sems[k % 2]
            cs = []
            for f in range(F):
                cs.append(
                    pltpu.async_copy(
                        table_hbm.at[idx_v.at[f, pl.ds(k * CHUNK, CHUNK)]],
                        buf.at[f],
                        sem,
                    )
                )
            return cs

        inflight = fire(0)
        for k in range(nchunks):
            buf = bufs[k % 2]
            nxt = fire(k + 1) if k + 1 < nchunks else []
            for c in inflight:
                c.wait()
            inflight = nxt

            def reduce_row(j, _):
                lo = buf[0, j, pl.ds(0, LANES)]
                hi = buf[0, j, pl.ds(LANES, LANES)]
                for f in range(1, F):
                    lo = lo + buf[f, j, pl.ds(0, LANES)]
                    hi = hi + buf[f, j, pl.ds(LANES, LANES)]
                out_v[j, pl.ds(0, LANES)] = lo
                out_v[j, pl.ds(LANES, LANES)] = hi
                return _

            lax.fori_loop(0, CHUNK, reduce_row, None)

            pltpu.sync_copy(out_v, out_hbm.at[pl.ds(b0 + k * CHUNK, CHUNK)])

    return emb_kernel


def kernel(categorical_feats, table):
    info = plsc.get_sparse_core_info()
    num_workers = info.num_cores * info.num_subcores
    feats_t = categorical_feats.T.astype(jnp.int32)
    return _make_kernel(num_workers)(feats_t, table)
